# Initial kernel scaffold; baseline (speedup 1.0000x reference)
import jax
import jax.numpy as jnp
from jax.experimental import pallas as pl


def kernel(indices, table):
    raise NotImplementedError("write your pallas kernel here")


# speedup vs baseline: 2.4198x; 2.4198x over previous
import functools

import jax
import jax.numpy as jnp
from jax import lax
from jax.experimental import pallas as pl
from jax.experimental.pallas import tpu as pltpu
from jax.experimental.pallas import tpu_sc as plsc

NC = 2
NS = 16
NW = NC * NS
LANE = 128
K = 8


def _gather_kernel(table, idx2d, out_rows, emb_dim):
    rows_total = idx2d.shape[0]
    rows_per_w = rows_total // NW
    mesh = plsc.VectorSubcoreMesh(core_axis_name="c", subcore_axis_name="s")

    @functools.partial(
        pl.kernel,
        mesh=mesh,
        out_type=out_rows,
        compiler_params=pltpu.CompilerParams(use_tc_tiling_on_sc=False),
        scratch_types=[
            pltpu.VMEM((K, LANE), jnp.int32),
            pltpu.VMEM((K * LANE, emb_dim), jnp.float32),
            pltpu.SemaphoreType.DMA,
        ],
    )
    def k(table_hbm, idx_hbm, out_hbm, idx_v, rows_v, sem):
        wid = lax.axis_index("s") * NC + lax.axis_index("c")
        row0 = wid * rows_per_w

        @pl.loop(0, rows_per_w, step=K)
        def _(r):
            base = row0 + r
            pltpu.sync_copy(idx_hbm.at[pl.ds(base, K)], idx_v)
            copies = []
            for j in range(K):
                copies.append(
                    pltpu.async_copy(
                        table_hbm.at[idx_v.at[j]],
                        rows_v.at[pl.ds(j * LANE, LANE)],
                        sem,
                    )
                )
            for c in copies:
                c.wait()
            pltpu.sync_copy(rows_v, out_hbm.at[pl.ds(base * LANE, K * LANE)])

    return k(table, idx2d)


def kernel(indices, table):
    n_rows, n_cols = indices.shape
    emb_dim = table.shape[1]
    total = n_rows * n_cols
    idx2d = indices.astype(jnp.int32).reshape(total // LANE, LANE)
    out_rows = jax.ShapeDtypeStruct((total, emb_dim), jnp.float32)
    out = _gather_kernel(table, idx2d, out_rows, emb_dim)
    return out.reshape(n_rows, n_cols, emb_dim)


# speedup vs baseline: 2.5309x; 1.0459x over previous
import functools

import jax
import jax.numpy as jnp
from jax import lax
from jax.experimental import pallas as pl
from jax.experimental.pallas import tpu as pltpu
from jax.experimental.pallas import tpu_sc as plsc

NC = 2
NS = 16
NW = NC * NS
LANE = 128
K = 16
NBUF = 2


def _gather_kernel(table, idx2d, out_rows, emb_dim):
    rows_total = idx2d.shape[0]
    rows_per_w = rows_total // NW
    steps = rows_per_w // K
    mesh = plsc.VectorSubcoreMesh(core_axis_name="c", subcore_axis_name="s")

    scratch = (
        [pltpu.VMEM((K, LANE), jnp.int32) for _ in range(NBUF)]
        + [pltpu.VMEM((K * LANE, emb_dim), jnp.float32) for _ in range(NBUF)]
        + [pltpu.SemaphoreType.DMA] * (3 * NBUF)
    )

    @functools.partial(
        pl.kernel,
        mesh=mesh,
        out_type=out_rows,
        compiler_params=pltpu.CompilerParams(use_tc_tiling_on_sc=False),
        scratch_types=scratch,
    )
    def k(table_hbm, idx_hbm, out_hbm, i0, i1, r0, r1, si0, si1, sg0, sg1,
          so0, so1):
        idx_v = [i0, i1]
        rows_v = [r0, r1]
        sem_i = [si0, si1]
        sem_g = [sg0, sg1]
        sem_o = [so0, so1]
        wid = lax.axis_index("s") * NC + lax.axis_index("c")
        row0 = wid * rows_per_w

        for b in range(NBUF):
            pltpu.async_copy(idx_hbm.at[pl.ds(row0 + b * K, K)], idx_v[b],
                             sem_i[b])

        @pl.loop(0, steps)
        def _(g):
            b = lax.rem(g, NBUF)

            def on_buf(bs):
                base = row0 + g * K

                @pl.when(g >= NBUF)
                def _():
                    pltpu.make_async_copy(
                        rows_v[bs], out_hbm.at[pl.ds(0, K * LANE)],
                        sem_o[bs]).wait()

                pltpu.make_async_copy(
                    idx_hbm.at[pl.ds(base, K)], idx_v[bs], sem_i[bs]).wait()

                for j in range(K):
                    pltpu.async_copy(
                        table_hbm.at[idx_v[bs].at[j]],
                        rows_v[bs].at[pl.ds(j * LANE, LANE)],
                        sem_g[bs],
                    )

                pltpu.make_async_copy(
                    table_hbm.at[pl.ds(0, K * LANE)], rows_v[bs],
                    sem_g[bs]).wait()

                @pl.when(g + NBUF < steps)
                def _():
                    pltpu.async_copy(
                        idx_hbm.at[pl.ds(base + NBUF * K, K)], idx_v[bs],
                        sem_i[bs])
                pltpu.async_copy(
                    rows_v[bs], out_hbm.at[pl.ds(base * LANE, K * LANE)],
                    sem_o[bs])

            @pl.when(b == 0)
            def _():
                on_buf(0)

            @pl.when(b == 1)
            def _():
                on_buf(1)

        for b in range(NBUF):
            pltpu.make_async_copy(
                rows_v[b], out_hbm.at[pl.ds(0, K * LANE)], sem_o[b]).wait()

    return k(table, idx2d)


def kernel(indices, table):
    n_rows, n_cols = indices.shape
    emb_dim = table.shape[1]
    total = n_rows * n_cols
    idx2d = indices.astype(jnp.int32).reshape(total // LANE, LANE)
    out_rows = jax.ShapeDtypeStruct((total, emb_dim), jnp.float32)
    out = _gather_kernel(table, idx2d, out_rows, emb_dim)
    return out.reshape(n_rows, n_cols, emb_dim)


# speedup vs baseline: 2.5310x; 1.0001x over previous
import functools

import jax
import jax.numpy as jnp
from jax import lax
from jax.experimental import pallas as pl
from jax.experimental.pallas import tpu as pltpu
from jax.experimental.pallas import tpu_sc as plsc

NC = 2
NS = 16
NW = NC * NS
CHUNK = 2048
NBUF = 2


def _gather_kernel(table, idx_flat, out_type, emb_dim):
    total = idx_flat.shape[0]
    per_w = total // NW
    steps = per_w // CHUNK
    mesh = plsc.VectorSubcoreMesh(core_axis_name="c", subcore_axis_name="s")

    scratch = (
        [pltpu.VMEM((CHUNK,), jnp.int32) for _ in range(NBUF)]
        + [pltpu.VMEM((CHUNK, emb_dim), jnp.float32) for _ in range(NBUF)]
        + [pltpu.SemaphoreType.DMA] * (3 * NBUF)
    )

    @functools.partial(
        pl.kernel,
        mesh=mesh,
        out_type=out_type,
        compiler_params=pltpu.CompilerParams(use_tc_tiling_on_sc=False),
        scratch_types=scratch,
    )
    def k(table_hbm, idx_hbm, out_hbm, i0, i1, r0, r1, si0, si1, sg0, sg1,
          so0, so1):
        idx_v = [i0, i1]
        rows_v = [r0, r1]
        sem_i = [si0, si1]
        sem_g = [sg0, sg1]
        sem_o = [so0, so1]
        wid = lax.axis_index("s") * NC + lax.axis_index("c")
        base0 = wid * per_w

        for b in range(NBUF):
            pltpu.async_copy(idx_hbm.at[pl.ds(base0 + b * CHUNK, CHUNK)],
                             idx_v[b], sem_i[b])

        @pl.loop(0, steps)
        def _(g):
            b = lax.rem(g, NBUF)

            def on_buf(bs):
                base = base0 + g * CHUNK

                @pl.when(g >= NBUF)
                def _():
                    pltpu.make_async_copy(
                        rows_v[bs], out_hbm.at[pl.ds(0, CHUNK)],
                        sem_o[bs]).wait()

                pltpu.make_async_copy(
                    idx_hbm.at[pl.ds(base, CHUNK)], idx_v[bs],
                    sem_i[bs]).wait()

                pltpu.async_copy(table_hbm.at[idx_v[bs]], rows_v[bs],
                                 sem_g[bs])

                pltpu.make_async_copy(
                    table_hbm.at[idx_v[bs]], rows_v[bs], sem_g[bs]).wait()

                @pl.when(g + NBUF < steps)
                def _():
                    pltpu.async_copy(
                        idx_hbm.at[pl.ds(base + NBUF * CHUNK, CHUNK)],
                        idx_v[bs], sem_i[bs])

                pltpu.async_copy(rows_v[bs], out_hbm.at[pl.ds(base, CHUNK)],
                                 sem_o[bs])

            @pl.when(b == 0)
            def _():
                on_buf(0)

            @pl.when(b == 1)
            def _():
                on_buf(1)

        for b in range(NBUF):
            pltpu.make_async_copy(
                rows_v[b], out_hbm.at[pl.ds(0, CHUNK)], sem_o[b]).wait()

    return k(table, idx_flat)


def kernel(indices, table):
    n_rows, n_cols = indices.shape
    emb_dim = table.shape[1]
    total = n_rows * n_cols
    idx_flat = indices.astype(jnp.int32).reshape(total)
    out2d = jax.ShapeDtypeStruct((total, emb_dim), jnp.float32)
    out = _gather_kernel(table, idx_flat, out2d, emb_dim)
    return out.reshape(n_rows, n_cols, emb_dim)


# speedup vs baseline: 2.5684x; 1.0148x over previous
import functools

import jax
import jax.numpy as jnp
from jax import lax
from jax.experimental import pallas as pl
from jax.experimental.pallas import tpu as pltpu
from jax.experimental.pallas import tpu_sc as plsc

NC = 2
NS = 16
NW = NC * NS
CHUNK = 2048
NBUF = 3


def _gather_kernel(table, idx_flat, out_type, emb_dim):
    total = idx_flat.shape[0]
    per_w = total // NW
    steps = per_w // CHUNK
    mesh = plsc.VectorSubcoreMesh(core_axis_name="c", subcore_axis_name="s")

    scratch = (
        [pltpu.VMEM((CHUNK,), jnp.int32) for _ in range(NBUF)]
        + [pltpu.VMEM((CHUNK, emb_dim), jnp.float32) for _ in range(NBUF)]
        + [pltpu.SemaphoreType.DMA] * (3 * NBUF)
    )

    @functools.partial(
        pl.kernel,
        mesh=mesh,
        out_type=out_type,
        compiler_params=pltpu.CompilerParams(use_tc_tiling_on_sc=False),
        scratch_types=scratch,
    )
    def k(table_hbm, idx_hbm, out_hbm, i0, i1, i2, r0, r1, r2,
          si0, si1, si2, sg0, sg1, sg2, so0, so1, so2):
        idx_v = [i0, i1, i2]
        rows_v = [r0, r1, r2]
        sem_i = [si0, si1, si2]
        sem_g = [sg0, sg1, sg2]
        sem_o = [so0, so1, so2]
        wid = lax.axis_index("s") * NC + lax.axis_index("c")
        base0 = wid * per_w

        def idx_load(chunk, b):
            pltpu.async_copy(idx_hbm.at[pl.ds(base0 + chunk * CHUNK, CHUNK)],
                             idx_v[b], sem_i[b])

        def idx_wait(b):
            pltpu.make_async_copy(idx_hbm.at[pl.ds(0, CHUNK)], idx_v[b],
                                  sem_i[b]).wait()

        def gather_fire(b):
            pltpu.async_copy(table_hbm.at[idx_v[b]], rows_v[b], sem_g[b])

        def gather_wait(b):
            pltpu.make_async_copy(table_hbm.at[idx_v[b]], rows_v[b],
                                  sem_g[b]).wait()

        def store_fire(chunk, b):
            pltpu.async_copy(rows_v[b],
                             out_hbm.at[pl.ds(base0 + chunk * CHUNK, CHUNK)],
                             sem_o[b])

        def store_wait(b):
            pltpu.make_async_copy(rows_v[b], out_hbm.at[pl.ds(0, CHUNK)],
                                  sem_o[b]).wait()

        idx_load(0, 0)
        idx_load(1, 1)
        idx_wait(0)
        gather_fire(0)

        @pl.loop(0, steps)
        def _(g):
            b = lax.rem(g, NBUF)

            def on_buf(bg):
                b1 = (bg + 1) % NBUF
                b2 = (bg + 2) % NBUF

                @pl.when(g + 1 < steps)
                def _():
                    idx_wait(b1)

                    @pl.when(g >= 2)
                    def _():
                        store_wait(b1)

                    gather_fire(b1)

                gather_wait(bg)
                store_fire(g, bg)

                @pl.when(g + 2 < steps)
                def _():
                    idx_load(g + 2, b2)

            for r in range(NBUF):
                @pl.when(b == r)
                def _(r=r):
                    on_buf(r)

        for b in range(NBUF):
            store_wait(b)

    return k(table, idx_flat)


def kernel(indices, table):
    n_rows, n_cols = indices.shape
    emb_dim = table.shape[1]
    total = n_rows * n_cols
    idx_flat = indices.astype(jnp.int32).reshape(total)
    out2d = jax.ShapeDtypeStruct((total, emb_dim), jnp.float32)
    out = _gather_kernel(table, idx_flat, out2d, emb_dim)
    return out.reshape(n_rows, n_cols, emb_dim)


# speedup vs baseline: 2.5689x; 1.0002x over previous
import functools

import jax
import jax.numpy as jnp
from jax import lax
from jax.experimental import pallas as pl
from jax.experimental.pallas import tpu as pltpu
from jax.experimental.pallas import tpu_sc as plsc

NC = 2
NS = 16
NW = NC * NS
CHUNK = 1024
NBUF = 4


def _gather_kernel(table, idx_flat, out_type, emb_dim):
    total = idx_flat.shape[0]
    per_w = total // NW
    steps = per_w // CHUNK
    mesh = plsc.VectorSubcoreMesh(core_axis_name="c", subcore_axis_name="s")

    scratch = (
        [pltpu.VMEM((CHUNK,), jnp.int32) for _ in range(NBUF)]
        + [pltpu.VMEM((CHUNK, emb_dim), jnp.float32) for _ in range(NBUF)]
        + [pltpu.SemaphoreType.DMA] * (3 * NBUF)
    )

    @functools.partial(
        pl.kernel,
        mesh=mesh,
        out_type=out_type,
        compiler_params=pltpu.CompilerParams(use_tc_tiling_on_sc=False),
        scratch_types=scratch,
    )
    def k(table_hbm, idx_hbm, out_hbm, i0, i1, i2, i3, r0, r1, r2, r3,
          si0, si1, si2, si3, sg0, sg1, sg2, sg3, so0, so1, so2, so3):
        idx_v = [i0, i1, i2, i3]
        rows_v = [r0, r1, r2, r3]
        sem_i = [si0, si1, si2, si3]
        sem_g = [sg0, sg1, sg2, sg3]
        sem_o = [so0, so1, so2, so3]
        wid = lax.axis_index("s") * NC + lax.axis_index("c")
        base0 = wid * per_w

        def idx_load(chunk, b):
            pltpu.async_copy(idx_hbm.at[pl.ds(base0 + chunk * CHUNK, CHUNK)],
                             idx_v[b], sem_i[b])

        def idx_wait(b):
            pltpu.make_async_copy(idx_hbm.at[pl.ds(0, CHUNK)], idx_v[b],
                                  sem_i[b]).wait()

        def gather_fire(b):
            pltpu.async_copy(table_hbm.at[idx_v[b]], rows_v[b], sem_g[b])

        def gather_wait(b):
            pltpu.make_async_copy(table_hbm.at[idx_v[b]], rows_v[b],
                                  sem_g[b]).wait()

        def store_fire(chunk, b):
            pltpu.async_copy(rows_v[b],
                             out_hbm.at[pl.ds(base0 + chunk * CHUNK, CHUNK)],
                             sem_o[b])

        def store_wait(b):
            pltpu.make_async_copy(rows_v[b], out_hbm.at[pl.ds(0, CHUNK)],
                                  sem_o[b]).wait()

        idx_load(0, 0)
        idx_load(1, 1)
        idx_load(2, 2)
        idx_wait(0)
        gather_fire(0)
        idx_wait(1)
        gather_fire(1)

        @pl.loop(0, steps)
        def _(g):
            b = lax.rem(g, NBUF)

            def on_buf(bg):
                b2 = (bg + 2) % NBUF
                b3 = (bg + 3) % NBUF

                @pl.when(g + 2 < steps)
                def _():
                    idx_wait(b2)

                    @pl.when(g >= 2)
                    def _():
                        store_wait(b2)

                    gather_fire(b2)

                gather_wait(bg)
                store_fire(g, bg)

                @pl.when(g + 3 < steps)
                def _():
                    idx_load(g + 3, b3)

            for r in range(NBUF):
                @pl.when(b == r)
                def _(r=r):
                    on_buf(r)

        for b in range(NBUF):
            store_wait(b)

    return k(table, idx_flat)


def kernel(indices, table):
    n_rows, n_cols = indices.shape
    emb_dim = table.shape[1]
    total = n_rows * n_cols
    idx_flat = indices.astype(jnp.int32).reshape(total)
    out2d = jax.ShapeDtypeStruct((total, emb_dim), jnp.float32)
    out = _gather_kernel(table, idx_flat, out2d, emb_dim)
    return out.reshape(n_rows, n_cols, emb_dim)
